# baseline (speedup 1.0000x reference)
import jax
import jax.numpy as jnp
from jax.experimental import pallas as pl
from jax.experimental.pallas import tpu as pltpu

NEG = -1e30
_B, _T, _V, _U = 16, 2048, 64, 256
_W = 384
_W2 = 768
_TB = 128
_NB = _T // _TB
_UF = 8


def _la2(a, b):
    m = jnp.maximum(a, b)
    return m + jnp.log1p(jnp.exp(jnp.minimum(a, b) - m))


def _ctc_kernel(logits_ref, targets_ref, loglen_ref, tgtlen_ref, out_ref,
                gs, g2, oh_scr, sem):
    lane = jax.lax.broadcasted_iota(jnp.int32, (_B, _W), 1)

    tgt = targets_ref[:, :]
    lane2 = jax.lax.broadcasted_iota(jnp.int32, (_B, _W2 - _U), 1)
    pad_cols = jnp.where(lane2 < _W - _U, -1, 0)
    tpad = jnp.concatenate([tgt, pad_cols], axis=1)

    iota_v = jax.lax.broadcasted_iota(jnp.int32, (_V, _W2), 0)
    for b in range(_B):
        row = jax.lax.broadcast_in_dim(tpad[b, :], (_V, _W2), (1,))
        oh_scr[b] = (iota_v == row).astype(jnp.float32)

    prev = jnp.concatenate(
        [jnp.full((_B, 1), -1, jnp.int32), tpad[:, :_W - 1]], axis=1)
    skip_mask = jnp.where(tpad[:, :_W] != prev, 0.0, NEG).astype(jnp.float32)

    loglen = loglen_ref[:, :]
    tgtlen = tgtlen_ref[:, :]

    def fill_block(blk, buf):
        t0 = blk * _TB
        for b in range(_B):
            a = logits_ref[b, pl.ds(t0, _TB), :]
            m = jnp.max(a, axis=1, keepdims=True)
            lse = jnp.log(jnp.sum(jnp.exp(a - m), axis=1, keepdims=True)) + m
            gb = jnp.dot(a, oh_scr[b], preferred_element_type=jnp.float32)
            gs[buf, b] = gb - lse

    def transpose_copies(buf):
        return [pltpu.make_async_copy(gs.at[buf, b], g2.at[buf, :, b, :],
                                      sem.at[buf]) for b in range(_B)]

    def start_transpose(buf):
        for c in transpose_copies(buf):
            c.start()

    def wait_transpose(buf):
        for c in transpose_copies(buf):
            c.wait()

    def read_g(buf, t_local):
        return g2[buf, pl.ds(t_local, 1), :, :].reshape(_B, _W2)

    def step(buf, t_local, t0, alpha_e, alpha_o, masked):
        g_t = read_g(buf, t_local)
        shift_o = jnp.concatenate(
            [jnp.full((_B, 1), NEG, jnp.float32), alpha_o[:, :-1]], axis=1)
        skip = shift_o + skip_mask
        m3 = jnp.maximum(jnp.maximum(alpha_o, alpha_e), skip)
        new_o = m3 + jnp.log(jnp.exp(alpha_o - m3) + jnp.exp(alpha_e - m3)
                             + jnp.exp(skip - m3)) + g_t[:, :_W]
        new_e = _la2(alpha_e, shift_o) + g_t[:, _W:]
        if masked:
            live = (t0 + t_local) < loglen
            return (jnp.where(live, new_e, alpha_e),
                    jnp.where(live, new_o, alpha_o))
        return new_e, new_o

    def run_block_steps(buf, t0, carry, masked, first):
        def inner(i, c):
            tl = first + i * _UF
            for k in range(_UF):
                c = step(buf, tl + k, t0, c[0], c[1], masked)
            return c
        if first:
            for k in range(1, _UF):
                carry = step(buf, k, t0, carry[0], carry[1], masked)
            return jax.lax.fori_loop(0, _TB // _UF - 1, inner, carry)
        return jax.lax.fori_loop(0, _TB // _UF, inner, carry)

    fill_block(0, 0)
    start_transpose(0)
    fill_block(1, 1)
    start_transpose(1)
    wait_transpose(0)

    g0 = read_g(0, 0)
    alpha_e = jnp.where(lane == 0, g0[:, _W:], NEG)
    alpha_o = jnp.where(lane == 0, g0[:, :_W], NEG)
    carry = run_block_steps(0, 0, (alpha_e, alpha_o), False, _UF)

    def body(k, c, masked):
        buf = jax.lax.rem(k, 2)
        nbuf = 1 - buf
        fill_block(k + 1, nbuf)
        start_transpose(nbuf)
        wait_transpose(buf)
        return run_block_steps(buf, k * _TB, c, masked, 0)

    carry = jax.lax.fori_loop(1, _NB // 2, lambda k, c: body(k, c, False),
                              carry)
    carry = jax.lax.fori_loop(_NB // 2, _NB - 1, lambda k, c: body(k, c, True),
                              carry)

    wait_transpose((_NB - 1) % 2)
    carry = run_block_steps((_NB - 1) % 2, (_NB - 1) * _TB, carry, True, 0)
    alpha_e, alpha_o = carry

    end1 = jnp.max(jnp.where(lane == tgtlen, alpha_e, NEG), axis=1,
                   keepdims=True)
    end2 = jnp.max(jnp.where(lane == tgtlen - 1, alpha_o, NEG), axis=1,
                   keepdims=True)
    end2 = jnp.where(tgtlen > 0, end2, NEG)
    ll = _la2(end1, end2)
    out_ref[:, :] = jnp.broadcast_to(-ll, (_B, 128))


def _run(logits, targets, loglen, tgtlen):
    return pl.pallas_call(
        _ctc_kernel,
        out_shape=jax.ShapeDtypeStruct((_B, 128), jnp.float32),
        scratch_shapes=[
            pltpu.VMEM((2, _B, _TB, _W2), jnp.float32),
            pltpu.VMEM((2, _TB, _B, _W2), jnp.float32),
            pltpu.VMEM((_B, _V, _W2), jnp.float32),
            pltpu.SemaphoreType.DMA((2,)),
        ],
    )(logits, targets, loglen, tgtlen)


@jax.jit
def kernel(logits, targets, logits_lengths, targets_lengths):
    loglen = logits_lengths.astype(jnp.int32).reshape(_B, 1)
    tgtlen = targets_lengths.astype(jnp.int32).reshape(_B, 1)
    out = _run(logits, targets.astype(jnp.int32), loglen, tgtlen)
    return out[:, 0]


# speedup vs baseline: 1.3223x; 1.3223x over previous
import jax
import jax.numpy as jnp
from jax.experimental import pallas as pl
from jax.experimental.pallas import tpu as pltpu

NEG = -1e30
_B, _T, _V, _U = 16, 2048, 64, 256
_W = 384
_TB = 128
_UF = 8
_TINY = 1e-37


def _ctc_kernel(logits_ref, targets_ref, loglen_ref, tgtlen_ref, out_ref,
                g_scr, oh_scr):
    lane = jax.lax.broadcasted_iota(jnp.int32, (_B, _W), 1)

    tgt = targets_ref[:, :]
    pad_col = jnp.where(
        jax.lax.broadcasted_iota(jnp.int32, (_B, _W - _U), 1) == 0, 0, -1)
    tpad = jnp.concatenate([tgt, pad_col], axis=1)

    iota_v = jax.lax.broadcasted_iota(jnp.int32, (_V, _W), 0)
    for b in range(_B):
        row = jax.lax.broadcast_in_dim(tpad[b, :], (_V, _W), (1,))
        oh_scr[b] = (iota_v == row).astype(jnp.float32)

    prev = jnp.concatenate(
        [jnp.full((_B, 1), -1, jnp.int32), tpad[:, :_W - 1]], axis=1)
    skip_mul = jnp.where(tpad != prev, 1.0, 0.0).astype(jnp.float32)

    loglen = loglen_ref[:, :]
    tgtlen = tgtlen_ref[:, :]

    def fill_block(blk):
        t0 = blk * _TB
        for b in range(_B):
            a = logits_ref[b, pl.ds(t0, _TB), :]
            m = jnp.max(a, axis=1, keepdims=True)
            lse = jnp.log(jnp.sum(jnp.exp(a - m), axis=1, keepdims=True)) + m
            gb = jnp.dot(a, oh_scr[b], preferred_element_type=jnp.float32)
            g_scr[b] = gb - lse

    def read_g(t_local):
        return g_scr[:, pl.ds(t_local, 1), :].reshape(_B, _W)

    def step(t_local, t0, alpha_e, alpha_o, masked):
        g_t = read_g(t_local)
        blank = jax.lax.broadcast_in_dim(g_t[:, _U], (_B, 1), (0,))
        shift_o = jnp.concatenate(
            [jnp.full((_B, 1), NEG, jnp.float32), alpha_o[:, :-1]], axis=1)
        m = jnp.maximum(jnp.maximum(alpha_o, alpha_e), shift_o)
        x_o = jnp.exp(alpha_o - m)
        x_e = jnp.exp(alpha_e - m)
        x_s = jnp.exp(shift_o - m)
        new_o = m + jnp.log(jnp.maximum(x_o + x_e + x_s * skip_mul,
                                        _TINY)) + g_t
        new_e = m + jnp.log(jnp.maximum(x_e + x_s, _TINY)) + blank
        if masked:
            live = (t0 + t_local) < loglen
            return (jnp.where(live, new_e, alpha_e),
                    jnp.where(live, new_o, alpha_o))
        return new_e, new_o

    def run_block_steps(t0, carry, masked, first):
        def inner(i, c):
            tl = first + i * _UF
            for k in range(_UF):
                c = step(tl + k, t0, c[0], c[1], masked)
            return c
        if first:
            for k in range(1, _UF):
                carry = step(k, t0, carry[0], carry[1], masked)
            return jax.lax.fori_loop(0, _TB // _UF - 1, inner, carry)
        return jax.lax.fori_loop(0, _TB // _UF, inner, carry)

    fill_block(0)
    g0 = read_g(0)
    blank0 = jax.lax.broadcast_in_dim(g0[:, _U], (_B, 1), (0,))
    alpha_e = jnp.where(lane == 0, jnp.broadcast_to(blank0, (_B, _W)), NEG)
    alpha_o = jnp.where(lane == 0, g0, NEG)
    carry = run_block_steps(0, (alpha_e, alpha_o), False, _UF)

    def block_body_live(blk, c):
        fill_block(blk)
        return run_block_steps(blk * _TB, c, False, 0)

    carry = jax.lax.fori_loop(1, _T // (2 * _TB), block_body_live, carry)

    def block_body_masked(blk, c):
        fill_block(blk)
        return run_block_steps(blk * _TB, c, True, 0)

    carry = jax.lax.fori_loop(_T // (2 * _TB), _T // _TB, block_body_masked,
                              carry)
    alpha_e, alpha_o = carry

    end1 = jnp.max(jnp.where(lane == tgtlen, alpha_e, NEG), axis=1,
                   keepdims=True)
    end2 = jnp.max(jnp.where(lane == tgtlen - 1, alpha_o, NEG), axis=1,
                   keepdims=True)
    end2 = jnp.where(tgtlen > 0, end2, NEG)
    mx = jnp.maximum(end1, end2)
    ll = mx + jnp.log(jnp.exp(end1 - mx) + jnp.exp(end2 - mx))
    out_ref[:, :] = jnp.broadcast_to(-ll, (_B, 128))


def _run(logits, targets, loglen, tgtlen):
    return pl.pallas_call(
        _ctc_kernel,
        out_shape=jax.ShapeDtypeStruct((_B, 128), jnp.float32),
        scratch_shapes=[
            pltpu.VMEM((_B, _TB, _W), jnp.float32),
            pltpu.VMEM((_B, _V, _W), jnp.float32),
        ],
    )(logits, targets, loglen, tgtlen)


@jax.jit
def kernel(logits, targets, logits_lengths, targets_lengths):
    loglen = logits_lengths.astype(jnp.int32).reshape(_B, 1)
    tgtlen = targets_lengths.astype(jnp.int32).reshape(_B, 1)
    out = _run(logits, targets.astype(jnp.int32), loglen, tgtlen)
    return out[:, 0]
